# baseline (speedup 1.0000x reference)
import functools

import jax
import jax.numpy as jnp
from jax import lax
from jax.experimental import pallas as pl
from jax.experimental.pallas import tpu as pltpu
from jax.experimental.pallas import tpu_sc as plsc

T = 2048
D = 768
H = 512
E = 64
TB = 64
NB = 96

_NC = 2
_NS = 16
_NW = _NC * _NS


def _router_body(x_ref, wr_ref, o_ref):
    scores = jnp.dot(x_ref[...], wr_ref[...],
                     preferred_element_type=jnp.float32)
    m = jnp.max(scores, axis=1, keepdims=True)
    ids = lax.broadcasted_iota(jnp.int32, scores.shape, 1)
    o_ref[...] = jnp.min(jnp.where(scores == m, ids, E), axis=1,
                         keepdims=True)


def _router(x2d, wr_t):
    return pl.pallas_call(
        _router_body,
        out_shape=jax.ShapeDtypeStruct((T, 1), jnp.int32),
    )(x2d, wr_t)


@functools.lru_cache(maxsize=None)
def _make_sc_gather(rows_table, rows_out, chunk):
    assert rows_out % (_NW * chunk) == 0 and chunk % 8 == 0
    n_chunks = rows_out // (_NW * chunk)
    mesh = plsc.VectorSubcoreMesh(core_axis_name="c", subcore_axis_name="s",
                                  num_cores=_NC, num_subcores=_NS)

    @functools.partial(
        pl.kernel,
        out_type=jax.ShapeDtypeStruct((rows_out, D), jnp.float32),
        mesh=mesh,
        scratch_types=(
            [pltpu.VMEM((chunk,), jnp.int32) for _ in range(n_chunks)]
            + [pltpu.VMEM((chunk, D), jnp.float32), pltpu.SemaphoreType.DMA]
        ),
    )
    def gather(table_hbm, idx_hbm, out_hbm, *scratch):
        idx_vs, rows_v, sem = scratch[:n_chunks], scratch[-2], scratch[-1]
        wid = lax.axis_index("s") * _NC + lax.axis_index("c")
        base = wid * n_chunks * chunk
        for c in range(n_chunks):
            pltpu.sync_copy(idx_hbm.at[pl.ds(base + c * chunk, chunk)],
                            idx_vs[c])
            pltpu.async_copy(table_hbm.at[idx_vs[c]], rows_v, sem).wait()
            pltpu.sync_copy(rows_v, out_hbm.at[pl.ds(base + c * chunk, chunk)])

    return gather


def _sc_gather(table, idx, chunk):
    return _make_sc_gather(table.shape[0], idx.shape[0], chunk)(table, idx)


def _ff_body(be_ref, nbt_ref, x_ref, w1_ref, w3_ref, w2_ref, o_ref):
    i = pl.program_id(0)

    @pl.when(i < nbt_ref[0])
    def _():
        xb = x_ref[...]
        a = jnp.dot(xb, w1_ref[0], preferred_element_type=jnp.float32)
        b = jnp.dot(xb, w3_ref[0], preferred_element_type=jnp.float32)
        h = a * jax.nn.sigmoid(a) * b
        o_ref[...] = jnp.dot(h, w2_ref[0], preferred_element_type=jnp.float32)


def _grouped_ff(be, nbt, x_ps, W1, W3, W2):
    grid_spec = pltpu.PrefetchScalarGridSpec(
        num_scalar_prefetch=2,
        grid=(NB,),
        in_specs=[
            pl.BlockSpec((TB, D), lambda i, be, nbt: (i, 0)),
            pl.BlockSpec((1, D, H), lambda i, be, nbt: (be[i], 0, 0)),
            pl.BlockSpec((1, D, H), lambda i, be, nbt: (be[i], 0, 0)),
            pl.BlockSpec((1, H, D), lambda i, be, nbt: (be[i], 0, 0)),
        ],
        out_specs=pl.BlockSpec((TB, D), lambda i, be, nbt: (i, 0)),
    )
    return pl.pallas_call(
        _ff_body,
        grid_spec=grid_spec,
        out_shape=jax.ShapeDtypeStruct((NB * TB, D), jnp.float32),
    )(be, nbt, x_ps, W1, W3, W2)


def kernel(x, W_router, W1, W2, W3):
    orig_shape = x.shape
    x2d = x.reshape(T, D)
    i32 = jnp.int32

    eid = _router(x2d, W_router.T)[:, 0]

    counts = jnp.zeros((E,), i32).at[eid].add(1)
    off = (jnp.cumsum(counts) - counts).astype(i32)
    nb = (counts + TB - 1) // TB
    inc = jnp.cumsum(nb).astype(i32)
    blk_start = inc - nb.astype(i32)
    total_blocks = inc[-1]

    i_arr = jnp.arange(NB, dtype=i32)
    be0 = jnp.sum((i_arr[:, None] >= inc[None, :]).astype(i32), axis=1)
    be0 = jnp.clip(be0, 0, E - 1)
    be_last = be0[jnp.maximum(total_blocks - 1, 0)]
    be = jnp.where(i_arr < total_blocks, be0, be_last)

    s_arr = jnp.arange(T, dtype=i32)
    e_s, order = lax.sort_key_val(eid, s_arr)
    is_new = jnp.concatenate([jnp.ones((1,), jnp.bool_),
                              e_s[1:] != e_s[:-1]])
    off_g = lax.cummax(jnp.where(is_new, s_arr, 0))
    onehot = (e_s[:, None] == jnp.arange(E, dtype=i32)[None, :])
    bs_g = jnp.dot(onehot.astype(jnp.float32), blk_start.astype(jnp.float32),
                   precision=lax.Precision.HIGHEST,
                   preferred_element_type=jnp.float32).astype(i32)
    p_s = bs_g * TB + (s_arr - off_g)
    gidx = (jnp.arange(NB * TB, dtype=i32) % T).at[p_s].set(order)
    pos = jnp.zeros((T,), i32).at[order].set(p_s)

    x_ps = _sc_gather(x2d, gidx, 96)
    y_ps = _grouped_ff(be, total_blocks[None], x_ps, W1, W3, W2)
    y = _sc_gather(y_ps, pos, TB)
    return y.reshape(orig_shape)


# speedup vs baseline: 1.1253x; 1.1253x over previous
import functools

import jax
import jax.numpy as jnp
from jax import lax
from jax.experimental import pallas as pl
from jax.experimental.pallas import tpu as pltpu
from jax.experimental.pallas import tpu_sc as plsc

T = 2048
D = 768
H = 512
E = 64
TB = 64
NB = 96

_NC = 2
_NS = 16
_NW = _NC * _NS


def _router_body(x_ref, wr_ref, o_ref):
    scores = jnp.dot(x_ref[...], wr_ref[...],
                     preferred_element_type=jnp.float32)
    m = jnp.max(scores, axis=1, keepdims=True)
    ids = lax.broadcasted_iota(jnp.int32, scores.shape, 1)
    o_ref[...] = jnp.min(jnp.where(scores == m, ids, E), axis=1,
                         keepdims=True)


def _router(x2d, wr_t):
    return pl.pallas_call(
        _router_body,
        out_shape=jax.ShapeDtypeStruct((T, 1), jnp.int32),
    )(x2d, wr_t)


@functools.lru_cache(maxsize=None)
def _make_sc_permute(rows_table, rows_out):
    chunk = T // _NW
    mesh = plsc.VectorSubcoreMesh(core_axis_name="c", subcore_axis_name="s",
                                  num_cores=_NC, num_subcores=_NS)

    @functools.partial(
        pl.kernel,
        out_type=jax.ShapeDtypeStruct((rows_out, D), jnp.float32),
        mesh=mesh,
        scratch_types=[
            pltpu.VMEM((chunk,), jnp.int32),
            pltpu.VMEM((chunk,), jnp.int32),
            pltpu.VMEM((chunk, D), jnp.float32),
            pltpu.SemaphoreType.DMA,
        ],
    )
    def permute(table_hbm, src_hbm, dst_hbm, out_hbm, src_v, dst_v, rows_v,
                sem):
        wid = lax.axis_index("s") * _NC + lax.axis_index("c")
        base = wid * chunk
        pltpu.sync_copy(src_hbm.at[pl.ds(base, chunk)], src_v)
        pltpu.sync_copy(dst_hbm.at[pl.ds(base, chunk)], dst_v)
        pltpu.async_copy(table_hbm.at[src_v], rows_v, sem).wait()
        pltpu.async_copy(rows_v, out_hbm.at[dst_v], sem).wait()

    return permute


def _sc_permute(table, src_idx, dst_idx, rows_out):
    return _make_sc_permute(table.shape[0], rows_out)(table, src_idx, dst_idx)


def _ff_body(be_ref, nbt_ref, x_ref, w1_ref, w3_ref, w2_ref, o_ref):
    i = pl.program_id(0)

    @pl.when(i < nbt_ref[0])
    def _():
        xb = x_ref[...]
        a = jnp.dot(xb, w1_ref[0], preferred_element_type=jnp.float32)
        b = jnp.dot(xb, w3_ref[0], preferred_element_type=jnp.float32)
        h = a * jax.nn.sigmoid(a) * b
        o_ref[...] = jnp.dot(h, w2_ref[0], preferred_element_type=jnp.float32)


def _grouped_ff(be, nbt, x_ps, W1, W3, W2):
    grid_spec = pltpu.PrefetchScalarGridSpec(
        num_scalar_prefetch=2,
        grid=(NB,),
        in_specs=[
            pl.BlockSpec((TB, D), lambda i, be, nbt: (i, 0)),
            pl.BlockSpec((1, D, H), lambda i, be, nbt: (be[i], 0, 0)),
            pl.BlockSpec((1, D, H), lambda i, be, nbt: (be[i], 0, 0)),
            pl.BlockSpec((1, H, D), lambda i, be, nbt: (be[i], 0, 0)),
        ],
        out_specs=pl.BlockSpec((TB, D), lambda i, be, nbt: (i, 0)),
    )
    return pl.pallas_call(
        _ff_body,
        grid_spec=grid_spec,
        out_shape=jax.ShapeDtypeStruct((NB * TB, D), jnp.float32),
    )(be, nbt, x_ps, W1, W3, W2)


def kernel(x, W_router, W1, W2, W3):
    orig_shape = x.shape
    x2d = x.reshape(T, D)
    i32 = jnp.int32

    eid = _router(x2d, W_router.T)[:, 0]

    s_arr = jnp.arange(T, dtype=i32)
    e_s, order = lax.sort_key_val(eid, s_arr)
    onehot = (e_s[:, None] == jnp.arange(E, dtype=i32)[None, :])
    counts = jnp.sum(onehot.astype(i32), axis=0)
    nb = (counts + TB - 1) // TB
    inc = jnp.cumsum(nb).astype(i32)
    blk_start = inc - nb.astype(i32)
    total_blocks = inc[-1]

    i_arr = jnp.arange(NB, dtype=i32)
    be0 = jnp.sum((i_arr[:, None] >= inc[None, :]).astype(i32), axis=1)
    be0 = jnp.clip(be0, 0, E - 1)
    be_last = be0[jnp.maximum(total_blocks - 1, 0)]
    be = jnp.where(i_arr < total_blocks, be0, be_last)

    is_new = jnp.concatenate([jnp.ones((1,), jnp.bool_),
                              e_s[1:] != e_s[:-1]])
    off_g = lax.cummax(jnp.where(is_new, s_arr, 0))
    bs_g = jnp.dot(onehot.astype(jnp.float32), blk_start.astype(jnp.float32),
                   precision=lax.Precision.HIGHEST,
                   preferred_element_type=jnp.float32).astype(i32)
    p_s = bs_g * TB + (s_arr - off_g)

    x_ps = _sc_permute(x2d, order, p_s, NB * TB)
    y_ps = _grouped_ff(be, total_blocks[None], x_ps, W1, W3, W2)
    y = _sc_permute(y_ps, p_s, order, T)
    return y.reshape(orig_shape)


# speedup vs baseline: 1.2424x; 1.1040x over previous
import functools

import jax
import jax.numpy as jnp
from jax import lax
from jax.experimental import pallas as pl
from jax.experimental.pallas import tpu as pltpu
from jax.experimental.pallas import tpu_sc as plsc

T = 2048
D = 768
H = 512
E = 64
TB = 64
NB = 96

_NC = 2
_NS = 16
_NW = _NC * _NS


def _router_body(x_ref, wr_ref, o_ref):
    scores = jnp.dot(x_ref[...], wr_ref[...],
                     preferred_element_type=jnp.float32)
    m = jnp.max(scores, axis=1, keepdims=True)
    ids = lax.broadcasted_iota(jnp.int32, scores.shape, 1)
    o_ref[...] = jnp.min(jnp.where(scores == m, ids, E), axis=1,
                         keepdims=True)


def _router(x2d, wr_t):
    return pl.pallas_call(
        _router_body,
        out_shape=jax.ShapeDtypeStruct((T, 1), jnp.int32),
    )(x2d, wr_t)


@functools.lru_cache(maxsize=None)
def _make_sc_permute(rows_table, rows_out):
    chunk = T // _NW
    mesh = plsc.VectorSubcoreMesh(core_axis_name="c", subcore_axis_name="s",
                                  num_cores=_NC, num_subcores=_NS)

    @functools.partial(
        pl.kernel,
        out_type=jax.ShapeDtypeStruct((rows_out, D), jnp.float32),
        mesh=mesh,
        scratch_types=[
            pltpu.VMEM((chunk,), jnp.int32),
            pltpu.VMEM((chunk,), jnp.int32),
            pltpu.VMEM((chunk, D), jnp.float32),
            pltpu.SemaphoreType.DMA,
        ],
    )
    def permute(table_hbm, src_hbm, dst_hbm, out_hbm, src_v, dst_v, rows_v,
                sem):
        wid = lax.axis_index("s") * _NC + lax.axis_index("c")
        base = wid * chunk
        pltpu.sync_copy(src_hbm.at[pl.ds(base, chunk)], src_v)
        pltpu.sync_copy(dst_hbm.at[pl.ds(base, chunk)], dst_v)
        pltpu.async_copy(table_hbm.at[src_v], rows_v, sem).wait()
        pltpu.async_copy(rows_v, out_hbm.at[dst_v], sem).wait()

    return permute


def _sc_permute(table, src_idx, dst_idx, rows_out):
    return _make_sc_permute(table.shape[0], rows_out)(table, src_idx, dst_idx)


def _ff_body(be_ref, nbt_ref, x_ref, w1_ref, w3_ref, w2_ref, o_ref):
    i = pl.program_id(0)

    @pl.when(i < nbt_ref[0])
    def _():
        xb = x_ref[...]
        a = jnp.dot(xb, w1_ref[0], preferred_element_type=jnp.float32)
        b = jnp.dot(xb, w3_ref[0], preferred_element_type=jnp.float32)
        h = a * jax.nn.sigmoid(a) * b
        o_ref[...] = jnp.dot(h, w2_ref[0], preferred_element_type=jnp.float32)


def _grouped_ff(be, nbt, x_ps, W1, W3, W2):
    grid_spec = pltpu.PrefetchScalarGridSpec(
        num_scalar_prefetch=2,
        grid=(NB,),
        in_specs=[
            pl.BlockSpec((TB, D),
                         lambda i, be, nbt: (jnp.minimum(i, nbt[0] - 1), 0)),
            pl.BlockSpec((1, D, H), lambda i, be, nbt: (be[i], 0, 0)),
            pl.BlockSpec((1, D, H), lambda i, be, nbt: (be[i], 0, 0)),
            pl.BlockSpec((1, H, D), lambda i, be, nbt: (be[i], 0, 0)),
        ],
        out_specs=pl.BlockSpec(
            (TB, D), lambda i, be, nbt: (jnp.minimum(i, nbt[0] - 1), 0)),
    )
    return pl.pallas_call(
        _ff_body,
        grid_spec=grid_spec,
        out_shape=jax.ShapeDtypeStruct((NB * TB, D), jnp.float32),
    )(be, nbt, x_ps, W1, W3, W2)


def kernel(x, W_router, W1, W2, W3):
    orig_shape = x.shape
    x2d = x.reshape(T, D)
    i32 = jnp.int32

    eid = _router(x2d, W_router.T)[:, 0]

    s_arr = jnp.arange(T, dtype=i32)
    e_s, order = lax.sort_key_val(eid, s_arr)
    onehot = (e_s[:, None] == jnp.arange(E, dtype=i32)[None, :])
    counts = jnp.sum(onehot.astype(i32), axis=0)
    nb = (counts + TB - 1) // TB
    inc = jnp.cumsum(nb).astype(i32)
    blk_start = inc - nb.astype(i32)
    total_blocks = inc[-1]

    i_arr = jnp.arange(NB, dtype=i32)
    be0 = jnp.sum((i_arr[:, None] >= inc[None, :]).astype(i32), axis=1)
    be0 = jnp.clip(be0, 0, E - 1)
    be_last = be0[jnp.maximum(total_blocks - 1, 0)]
    be = jnp.where(i_arr < total_blocks, be0, be_last)

    is_new = jnp.concatenate([jnp.ones((1,), jnp.bool_),
                              e_s[1:] != e_s[:-1]])
    off_g = lax.cummax(jnp.where(is_new, s_arr, 0))
    bs_g = jnp.dot(onehot.astype(jnp.float32), blk_start.astype(jnp.float32),
                   precision=lax.Precision.HIGHEST,
                   preferred_element_type=jnp.float32).astype(i32)
    p_s = bs_g * TB + (s_arr - off_g)

    x_ps = _sc_permute(x2d, order, p_s, NB * TB)
    y_ps = _grouped_ff(be, total_blocks[None], x_ps, W1, W3, W2)
    y = _sc_permute(y_ps, p_s, order, T)
    return y.reshape(orig_shape)
